# Initial kernel scaffold; baseline (speedup 1.0000x reference)
import jax
import jax.numpy as jnp
from jax.experimental import pallas as pl


def kernel(x, edge_index, W1, att_src1, att_dst1, b1, W2, att_src2, att_dst2, b2):
    raise NotImplementedError("write your pallas kernel here")


# speedup vs baseline: 22.9368x; 22.9368x over previous
import jax
import jax.numpy as jnp
from jax import lax
from jax.experimental import pallas as pl
from jax.experimental.pallas import tpu as pltpu
from jax.experimental.pallas import tpu_sc as plsc

N_NODES = 10000
N_EDGES = 320000
D_FEAT = 128
H1 = 8
O1 = 8
C2 = 47

NC = 2
NS = 16
NW = NC * NS

RW = 128

R_BLK = 400
N_BLKS = N_NODES // R_BLK

EPW = N_EDGES // NW
B_E = 80
N_CH = EPW // B_E

ROW_STRIDE = 624
ROW_COPIES = 8


def _tc_a_body(x_ref, w1_ref, as_ref, ad_ref, t1_ref, mx_ref):
    h = jnp.dot(x_ref[...], w1_ref[...], preferred_element_type=jnp.float32)
    asrc = jnp.dot(h, as_ref[...], preferred_element_type=jnp.float32)
    adst = jnp.dot(h, ad_ref[...], preferred_element_type=jnp.float32)
    z48 = jnp.zeros((asrc.shape[0], 48), jnp.float32)
    t1_ref[...] = jnp.concatenate([asrc, adst, h, z48], axis=1)
    m8 = jnp.broadcast_to(jnp.max(asrc, axis=0)[None, :], (8, 8))

    @pl.when(pl.program_id(0) == 0)
    def _():
        mx_ref[...] = m8

    @pl.when(pl.program_id(0) != 0)
    def _():
        mx_ref[...] = jnp.maximum(mx_ref[...], m8)


def _tc_a(x, w1, a_src_m, a_dst_m):
    return pl.pallas_call(
        _tc_a_body,
        grid=(N_BLKS,),
        in_specs=[
            pl.BlockSpec((R_BLK, D_FEAT), lambda i: (i, 0)),
            pl.BlockSpec((D_FEAT, H1 * O1), lambda i: (0, 0)),
            pl.BlockSpec((H1 * O1, 8), lambda i: (0, 0)),
            pl.BlockSpec((H1 * O1, 8), lambda i: (0, 0)),
        ],
        out_specs=[
            pl.BlockSpec((R_BLK, RW), lambda i: (i, 0)),
            pl.BlockSpec((8, 8), lambda i: (0, 0)),
        ],
        out_shape=[
            jax.ShapeDtypeStruct((N_NODES, RW), jnp.float32),
            jax.ShapeDtypeStruct((8, 8), jnp.float32),
        ],
    )(x, w1, a_src_m, a_dst_m)


def _make_sc_edge(heads, oc):
    mesh = plsc.VectorSubcoreMesh(
        core_axis_name="c", subcore_axis_name="s", num_cores=NC, num_subcores=NS
    )

    def body(t_hbm, src_hbm, dst_hbm, mx_hbm, out_hbm,
             accum, srows, drows, sidx, didx, mxv, sem1, sem2):
        c = lax.axis_index("c")
        s = lax.axis_index("s")
        wid = c * NS + s

        pltpu.sync_copy(mx_hbm, mxv)

        zero16 = jnp.zeros((16,), jnp.float32)

        def zrow(i, carry):
            for k in range(RW // 16):
                srows[i, pl.ds(k * 16, 16)] = zero16
            return carry

        lax.fori_loop(0, B_E, zrow, 0)

        row0 = s * ROW_STRIDE
        for j in range(ROW_COPIES):
            pltpu.sync_copy(srows.at[pl.ds(0, B_E)],
                            accum.at[pl.ds(row0 + j * B_E, B_E)])
        plsc.subcore_barrier()

        iota = lax.iota(jnp.int32, 16)
        mrows = [mxv[h, pl.ds(0, 16)] for h in range(heads)]

        def chunk(ci, carry):
            base = wid * EPW + ci * B_E
            pltpu.sync_copy(src_hbm.at[pl.ds(base, B_E)], sidx)
            pltpu.sync_copy(dst_hbm.at[pl.ds(base, B_E)], didx)
            cp1 = pltpu.async_copy(t_hbm.at[sidx], srows, sem1)
            cp2 = pltpu.async_copy(t_hbm.at[didx], drows, sem2)
            cp1.wait()
            cp2.wait()
            for g in range(B_E // 16):
                rowid = g * 16 + iota
                es = []
                for h in range(heads):
                    colh = jnp.full((16,), h, jnp.int32)
                    cold = jnp.full((16,), 8 + h, jnp.int32)
                    a_s = plsc.load_gather(srows, [rowid, colh])
                    a_d = plsc.load_gather(drows, [rowid, cold])
                    sv = a_s + a_d
                    lr = jnp.where(sv >= 0.0, sv, 0.2 * sv)
                    cv = jnp.maximum(mrows[h] + a_d, 0.0)
                    e_h = jnp.exp(lr - cv)
                    plsc.store_scatter(srows, [rowid, colh], e_h)
                    es.append(e_h)
                for h in range(heads):
                    for o in range(oc):
                        colc = jnp.full((16,), 16 + h * oc + o, jnp.int32)
                        hv = plsc.load_gather(srows, [rowid, colc])
                        plsc.store_scatter(srows, [rowid, colc], hv * es[h])
            pltpu.sync_copy(srows, accum.at[didx], add=True)
            return carry

        lax.fori_loop(0, N_CH, chunk, 0)
        plsc.subcore_barrier()

        for j in range(ROW_COPIES):
            off = row0 + j * B_E
            pltpu.sync_copy(accum.at[pl.ds(off, B_E)],
                            out_hbm.at[c, pl.ds(off, B_E)])

    return pl.kernel(
        body,
        out_type=jax.ShapeDtypeStruct((NC, N_NODES, RW), jnp.float32),
        mesh=mesh,
        scratch_types=[
            pltpu.VMEM_SHARED((N_NODES, RW), jnp.float32),
            pltpu.VMEM((B_E, RW), jnp.float32),
            pltpu.VMEM((B_E, RW), jnp.float32),
            pltpu.VMEM((B_E,), jnp.int32),
            pltpu.VMEM((B_E,), jnp.int32),
            pltpu.VMEM((heads, RW), jnp.float32),
            pltpu.SemaphoreType.DMA,
            pltpu.SemaphoreType.DMA,
        ],
        compiler_params=pltpu.CompilerParams(needs_layout_passes=False),
    )


def _tc_b1_body(t1_ref, ac0_ref, ac1_ref, mx_ref, b1_ref, rep_ref,
                w2_ref, s2_ref, d2_ref, t2_ref, mx2_ref):
    asrc = t1_ref[:, 0:8]
    adst = t1_ref[:, 8:16]
    h1 = t1_ref[:, 16:80]
    sv = asrc + adst
    lr = jnp.where(sv >= 0.0, sv, 0.2 * sv)
    cv = jnp.maximum(mx_ref[0:1, :] + adst, 0.0)
    e = jnp.exp(lr - cv)
    denom = ac0_ref[:, 0:8] + ac1_ref[:, 0:8] + e
    e64 = jnp.dot(e, rep_ref[...], preferred_element_type=jnp.float32)
    den64 = jnp.dot(denom, rep_ref[...], preferred_element_type=jnp.float32)
    msg = ac0_ref[:, 16:80] + ac1_ref[:, 16:80] + h1 * e64
    out1 = msg / den64 + b1_ref[0:1, :]
    out1 = jnp.maximum(out1, 0.0)
    h2 = jnp.dot(out1, w2_ref[...], preferred_element_type=jnp.float32)
    a2s = jnp.dot(h2, s2_ref[...], preferred_element_type=jnp.float32)
    a2d = jnp.dot(h2, d2_ref[...], preferred_element_type=jnp.float32)
    n = h2.shape[0]
    z7 = jnp.zeros((n, 7), jnp.float32)
    z64 = jnp.zeros((n, 64), jnp.float32)
    t2_ref[...] = jnp.concatenate(
        [a2s[:, 0:1], z7, a2d[:, 0:1], z7, h2, z64], axis=1)
    m8 = jnp.full((8, 8), jnp.max(a2s[:, 0]), jnp.float32)

    @pl.when(pl.program_id(0) == 0)
    def _():
        mx2_ref[...] = m8

    @pl.when(pl.program_id(0) != 0)
    def _():
        mx2_ref[...] = jnp.maximum(mx2_ref[...], m8)


def _tc_b1(t1, ac0, ac1, mx1, b1b, rep, w2p, s2, d2):
    return pl.pallas_call(
        _tc_b1_body,
        grid=(N_BLKS,),
        in_specs=[
            pl.BlockSpec((R_BLK, RW), lambda i: (i, 0)),
            pl.BlockSpec((R_BLK, RW), lambda i: (i, 0)),
            pl.BlockSpec((R_BLK, RW), lambda i: (i, 0)),
            pl.BlockSpec((8, 8), lambda i: (0, 0)),
            pl.BlockSpec((8, 64), lambda i: (0, 0)),
            pl.BlockSpec((8, 64), lambda i: (0, 0)),
            pl.BlockSpec((64, 48), lambda i: (0, 0)),
            pl.BlockSpec((48, 8), lambda i: (0, 0)),
            pl.BlockSpec((48, 8), lambda i: (0, 0)),
        ],
        out_specs=[
            pl.BlockSpec((R_BLK, RW), lambda i: (i, 0)),
            pl.BlockSpec((8, 8), lambda i: (0, 0)),
        ],
        out_shape=[
            jax.ShapeDtypeStruct((N_NODES, RW), jnp.float32),
            jax.ShapeDtypeStruct((8, 8), jnp.float32),
        ],
    )(t1, ac0, ac1, mx1, b1b, rep, w2p, s2, d2)


def _tc_b2_body(t2_ref, ac0_ref, ac1_ref, mx_ref, b2_ref, o_ref):
    a2 = t2_ref[:, 0:1]
    ad = t2_ref[:, 8:9]
    h2 = t2_ref[:, 16:64]
    sv = a2 + ad
    lr = jnp.where(sv >= 0.0, sv, 0.2 * sv)
    cv = jnp.maximum(mx_ref[0:1, 0:1] + ad, 0.0)
    e = jnp.exp(lr - cv)
    den = ac0_ref[:, 0:1] + ac1_ref[:, 0:1] + e
    msg = ac0_ref[:, 16:64] + ac1_ref[:, 16:64] + h2 * e
    out = msg / den + b2_ref[0:1, :]
    col = lax.broadcasted_iota(jnp.int32, out.shape, 1)
    valid = col < C2
    z = jnp.where(valid, out, -jnp.inf)
    m = jnp.max(z, axis=1, keepdims=True)
    ssum = jnp.sum(jnp.where(valid, jnp.exp(z - m), 0.0), axis=1, keepdims=True)
    o_ref[...] = z - m - jnp.log(ssum)


def _tc_b2(t2, ac0, ac1, mx2, b2b):
    return pl.pallas_call(
        _tc_b2_body,
        grid=(N_BLKS,),
        in_specs=[
            pl.BlockSpec((R_BLK, RW), lambda i: (i, 0)),
            pl.BlockSpec((R_BLK, RW), lambda i: (i, 0)),
            pl.BlockSpec((R_BLK, RW), lambda i: (i, 0)),
            pl.BlockSpec((8, 8), lambda i: (0, 0)),
            pl.BlockSpec((8, 48), lambda i: (0, 0)),
        ],
        out_specs=pl.BlockSpec((R_BLK, 48), lambda i: (i, 0)),
        out_shape=jax.ShapeDtypeStruct((N_NODES, 48), jnp.float32),
    )(t2, ac0, ac1, mx2, b2b)


_sc_edge_1 = _make_sc_edge(H1, O1)
_sc_edge_2 = _make_sc_edge(1, C2)


def kernel(x, edge_index, W1, att_src1, att_dst1, b1, W2, att_src2, att_dst2, b2):
    src = edge_index[0]
    dst = edge_index[1]

    eye8 = jnp.eye(8, dtype=jnp.float32)
    a_src_m = (att_src1[:, :, None] * eye8[:, None, :]).reshape(64, 8)
    a_dst_m = (att_dst1[:, :, None] * eye8[:, None, :]).reshape(64, 8)
    rep = jnp.repeat(eye8, 8, axis=1)
    w2p = jnp.pad(W2, ((0, 0), (0, 1)))
    s2 = jnp.pad(att_src2.reshape(C2, 1), ((0, 1), (0, 7)))
    d2 = jnp.pad(att_dst2.reshape(C2, 1), ((0, 1), (0, 7)))
    b1b = jnp.broadcast_to(b1[None, :], (8, 64))
    b2b = jnp.broadcast_to(jnp.pad(b2, (0, 1))[None, :], (8, 48))

    t1, mx1 = _tc_a(x, W1, a_src_m, a_dst_m)
    mx1_bk = jnp.broadcast_to(mx1[0][:, None], (H1, RW))
    acc1 = _sc_edge_1(t1, src, dst, mx1_bk)
    t2, mx2 = _tc_b1(t1, acc1[0], acc1[1], mx1, b1b, rep, w2p, s2, d2)
    mx2_bk = jnp.broadcast_to(mx2[0:1, 0:1], (1, RW))
    acc2 = _sc_edge_2(t2, src, dst, mx2_bk)
    out = _tc_b2(t2, acc2[0], acc2[1], mx2, b2b)
    return out[:, :C2]


# speedup vs baseline: 29.8954x; 1.3034x over previous
import jax
import jax.numpy as jnp
from jax import lax
from jax.experimental import pallas as pl
from jax.experimental.pallas import tpu as pltpu
from jax.experimental.pallas import tpu_sc as plsc

N_NODES = 10000
N_EDGES = 320000
D_FEAT = 128
H1 = 8
O1 = 8
C2 = 47

NC = 2
NS = 16
NW = NC * NS

RW = 128

R_BLK = 400
N_BLKS = N_NODES // R_BLK

EPW = N_EDGES // NW
B_E = 80
N_CH = EPW // B_E

ROW_STRIDE = 624
ROW_COPIES = 8


def _tc_a_body(x_ref, w1_ref, as_ref, ad_ref, t1_ref, mx_ref):
    h = jnp.dot(x_ref[...], w1_ref[...], preferred_element_type=jnp.float32)
    asrc = jnp.dot(h, as_ref[...], preferred_element_type=jnp.float32)
    adst = jnp.dot(h, ad_ref[...], preferred_element_type=jnp.float32)
    z48 = jnp.zeros((asrc.shape[0], 48), jnp.float32)
    t1_ref[...] = jnp.concatenate([asrc, adst, h, z48], axis=1)
    m8 = jnp.broadcast_to(jnp.max(asrc, axis=0)[None, :], (8, 8))

    @pl.when(pl.program_id(0) == 0)
    def _():
        mx_ref[...] = m8

    @pl.when(pl.program_id(0) != 0)
    def _():
        mx_ref[...] = jnp.maximum(mx_ref[...], m8)


def _tc_a(x, w1, a_src_m, a_dst_m):
    return pl.pallas_call(
        _tc_a_body,
        grid=(N_BLKS,),
        in_specs=[
            pl.BlockSpec((R_BLK, D_FEAT), lambda i: (i, 0)),
            pl.BlockSpec((D_FEAT, H1 * O1), lambda i: (0, 0)),
            pl.BlockSpec((H1 * O1, 8), lambda i: (0, 0)),
            pl.BlockSpec((H1 * O1, 8), lambda i: (0, 0)),
        ],
        out_specs=[
            pl.BlockSpec((R_BLK, RW), lambda i: (i, 0)),
            pl.BlockSpec((8, 8), lambda i: (0, 0)),
        ],
        out_shape=[
            jax.ShapeDtypeStruct((N_NODES, RW), jnp.float32),
            jax.ShapeDtypeStruct((8, 8), jnp.float32),
        ],
    )(x, w1, a_src_m, a_dst_m)


def _make_sc_edge(heads, oc, adt_rows, dshift, dmask, aw, mo):
    mesh = plsc.VectorSubcoreMesh(
        core_axis_name="c", subcore_axis_name="s", num_cores=NC, num_subcores=NS
    )

    def body(t_hbm, src_hbm, dst_hbm, adp_hbm, mx_hbm, out_hbm,
             accum, adt, srows0, srows1, drows0, drows1, mbuf,
             sp0, sp1, didx, mxv, sem0, sem1, sem2, sem3, semi0, semi1):
        c = lax.axis_index("c")
        s = lax.axis_index("s")
        wid = c * NS + s

        pltpu.sync_copy(mx_hbm, mxv)
        pltpu.sync_copy(dst_hbm.at[wid], didx)

        @pl.when(s == 0)
        def _():
            pltpu.sync_copy(adp_hbm, adt)

        zero16 = jnp.zeros((16,), jnp.float32)

        def zrow(i, carry):
            for k in range(aw // 16):
                mbuf[i, pl.ds(k * 16, 16)] = zero16
            return carry

        lax.fori_loop(0, B_E, zrow, 0)

        row0 = s * ROW_STRIDE
        for j in range(ROW_COPIES):
            pltpu.sync_copy(mbuf.at[pl.ds(0, B_E)],
                            accum.at[pl.ds(row0 + j * B_E, B_E)])
        plsc.subcore_barrier()

        iota = lax.iota(jnp.int32, 16)
        mrows = [mxv[h, pl.ds(0, 16)] for h in range(heads)]

        def fetch_idx(ci, sp, semi):
            cif = jnp.minimum(ci, N_CH - 1)
            pltpu.async_copy(src_hbm.at[wid, cif], sp, semi)

        def wait_idx(sp, semi):
            pltpu.make_async_copy(src_hbm.at[wid, 0], sp, semi).wait()

        def gathers(ci, sp, sbuf, dbuf, sem_s, sem_d):
            pltpu.async_copy(t_hbm.at[sp], sbuf, sem_s)
            for g in range(B_E // 16):
                dvec = didx[ci, pl.ds(g * 16, 16)]
                drow = lax.shift_right_logical(dvec, dshift)
                pltpu.async_copy(adt.at[drow], dbuf.at[pl.ds(g * 16, 16)],
                                 sem_d)

        def wait_gathers(sp, sbuf, dbuf, sem_s, sem_d):
            pltpu.make_async_copy(t_hbm.at[sp], sbuf, sem_s).wait()
            pltpu.make_async_copy(t_hbm.at[sp], dbuf, sem_d).wait()

        def compute_and_scatter(ci, buf, dbuf):
            for g in range(B_E // 16):
                rowid = g * 16 + iota
                dv = didx[ci, pl.ds(g * 16, 16)]
                dcol = (dv & dmask) * heads
                es = []
                for h in range(heads):
                    colh = jnp.full((16,), h, jnp.int32)
                    a_s = plsc.load_gather(buf, [rowid, colh])
                    a_d = plsc.load_gather(dbuf, [rowid, dcol + h])
                    sv = a_s + a_d
                    lr = jnp.where(sv >= 0.0, sv, 0.2 * sv)
                    cv = jnp.maximum(mrows[h] + a_d, 0.0)
                    e_h = jnp.exp(lr - cv)
                    plsc.store_scatter(mbuf, [rowid, colh], e_h)
                    es.append(e_h)
                for h in range(heads):
                    for o in range(oc):
                        colc = jnp.full((16,), mo + h * oc + o, jnp.int32)
                        hv = plsc.load_gather(buf, [rowid, colc])
                        plsc.store_scatter(mbuf, [rowid, colc], hv * es[h])
            pltpu.sync_copy(mbuf, accum.at[didx.at[ci]], add=True)

        pltpu.sync_copy(src_hbm.at[wid, 0], sp0)
        gathers(0, sp0, srows0, drows0, sem0, sem2)
        fetch_idx(1, sp1, semi1)
        wait_gathers(sp0, srows0, drows0, sem0, sem2)

        def pair(k, carry):
            ci0 = 2 * k
            wait_idx(sp1, semi1)
            gathers(ci0 + 1, sp1, srows1, drows1, sem1, sem3)
            fetch_idx(ci0 + 2, sp0, semi0)
            compute_and_scatter(ci0, srows0, drows0)
            wait_gathers(sp1, srows1, drows1, sem1, sem3)
            wait_idx(sp0, semi0)
            gathers(ci0 + 2, sp0, srows0, drows0, sem0, sem2)
            fetch_idx(ci0 + 3, sp1, semi1)
            compute_and_scatter(ci0 + 1, srows1, drows1)
            wait_gathers(sp0, srows0, drows0, sem0, sem2)
            return carry

        lax.fori_loop(0, (N_CH - 1) // 2, pair, 0)
        wait_idx(sp1, semi1)
        compute_and_scatter(N_CH - 1, srows0, drows0)
        plsc.subcore_barrier()

        for j in range(ROW_COPIES):
            off = row0 + j * B_E
            pltpu.sync_copy(accum.at[pl.ds(off, B_E)],
                            out_hbm.at[c, pl.ds(off, B_E)])

    return pl.kernel(
        body,
        out_type=jax.ShapeDtypeStruct((NC, N_NODES, aw), jnp.float32),
        mesh=mesh,
        scratch_types=[
            pltpu.VMEM_SHARED((N_NODES, aw), jnp.float32),
            pltpu.VMEM_SHARED((adt_rows, RW), jnp.float32),
            pltpu.VMEM((B_E, RW), jnp.float32),
            pltpu.VMEM((B_E, RW), jnp.float32),
            pltpu.VMEM((B_E, RW), jnp.float32),
            pltpu.VMEM((B_E, RW), jnp.float32),
            pltpu.VMEM((B_E, aw), jnp.float32),
            pltpu.VMEM((B_E,), jnp.int32),
            pltpu.VMEM((B_E,), jnp.int32),
            pltpu.VMEM((N_CH, B_E), jnp.int32),
            pltpu.VMEM((heads, RW), jnp.float32),
            pltpu.SemaphoreType.DMA,
            pltpu.SemaphoreType.DMA,
            pltpu.SemaphoreType.DMA,
            pltpu.SemaphoreType.DMA,
            pltpu.SemaphoreType.DMA,
            pltpu.SemaphoreType.DMA,
        ],
        compiler_params=pltpu.CompilerParams(needs_layout_passes=False),
    )


def _tc_b1_body(t1_ref, ac0_ref, ac1_ref, mx_ref, b1_ref, rep_ref,
                w2_ref, s2_ref, d2_ref, t2_ref, mx2_ref):
    asrc = t1_ref[:, 0:8]
    adst = t1_ref[:, 8:16]
    h1 = t1_ref[:, 16:80]
    sv = asrc + adst
    lr = jnp.where(sv >= 0.0, sv, 0.2 * sv)
    cv = jnp.maximum(mx_ref[0:1, :] + adst, 0.0)
    e = jnp.exp(lr - cv)
    denom = ac0_ref[:, 0:8] + ac1_ref[:, 0:8] + e
    e64 = jnp.dot(e, rep_ref[...], preferred_element_type=jnp.float32)
    den64 = jnp.dot(denom, rep_ref[...], preferred_element_type=jnp.float32)
    msg = ac0_ref[:, 8:72] + ac1_ref[:, 8:72] + h1 * e64
    out1 = msg / den64 + b1_ref[0:1, :]
    out1 = jnp.maximum(out1, 0.0)
    h2 = jnp.dot(out1, w2_ref[...], preferred_element_type=jnp.float32)
    a2s = jnp.dot(h2, s2_ref[...], preferred_element_type=jnp.float32)
    a2d = jnp.dot(h2, d2_ref[...], preferred_element_type=jnp.float32)
    n = h2.shape[0]
    z7 = jnp.zeros((n, 7), jnp.float32)
    z64 = jnp.zeros((n, 64), jnp.float32)
    t2_ref[...] = jnp.concatenate(
        [a2s[:, 0:1], z7, a2d[:, 0:1], z7, h2, z64], axis=1)
    m8 = jnp.full((8, 8), jnp.max(a2s[:, 0]), jnp.float32)

    @pl.when(pl.program_id(0) == 0)
    def _():
        mx2_ref[...] = m8

    @pl.when(pl.program_id(0) != 0)
    def _():
        mx2_ref[...] = jnp.maximum(mx2_ref[...], m8)


def _tc_b1(t1, ac0, ac1, mx1, b1b, rep, w2p, s2, d2):
    return pl.pallas_call(
        _tc_b1_body,
        grid=(N_BLKS,),
        in_specs=[
            pl.BlockSpec((R_BLK, RW), lambda i: (i, 0)),
            pl.BlockSpec((R_BLK, 72), lambda i: (i, 0)),
            pl.BlockSpec((R_BLK, 72), lambda i: (i, 0)),
            pl.BlockSpec((8, 8), lambda i: (0, 0)),
            pl.BlockSpec((8, 64), lambda i: (0, 0)),
            pl.BlockSpec((8, 64), lambda i: (0, 0)),
            pl.BlockSpec((64, 48), lambda i: (0, 0)),
            pl.BlockSpec((48, 8), lambda i: (0, 0)),
            pl.BlockSpec((48, 8), lambda i: (0, 0)),
        ],
        out_specs=[
            pl.BlockSpec((R_BLK, RW), lambda i: (i, 0)),
            pl.BlockSpec((8, 8), lambda i: (0, 0)),
        ],
        out_shape=[
            jax.ShapeDtypeStruct((N_NODES, RW), jnp.float32),
            jax.ShapeDtypeStruct((8, 8), jnp.float32),
        ],
    )(t1, ac0, ac1, mx1, b1b, rep, w2p, s2, d2)


def _tc_b2_body(t2_ref, ac0_ref, ac1_ref, mx_ref, b2_ref, o_ref):
    a2 = t2_ref[:, 0:1]
    ad = t2_ref[:, 8:9]
    h2 = t2_ref[:, 16:63]
    sv = a2 + ad
    lr = jnp.where(sv >= 0.0, sv, 0.2 * sv)
    cv = jnp.maximum(mx_ref[0:1, 0:1] + ad, 0.0)
    e = jnp.exp(lr - cv)
    den = ac0_ref[:, 0:1] + ac1_ref[:, 0:1] + e
    z = (ac0_ref[:, 1:48] + ac1_ref[:, 1:48] + h2 * e) / den + b2_ref[0:1, :]
    m = jnp.max(z, axis=1, keepdims=True)
    ssum = jnp.sum(jnp.exp(z - m), axis=1, keepdims=True)
    o_ref[...] = z - m - jnp.log(ssum)


def _tc_b2(t2, ac0, ac1, mx2, b2b):
    return pl.pallas_call(
        _tc_b2_body,
        grid=(N_BLKS,),
        in_specs=[
            pl.BlockSpec((R_BLK, RW), lambda i: (i, 0)),
            pl.BlockSpec((R_BLK, 48), lambda i: (i, 0)),
            pl.BlockSpec((R_BLK, 48), lambda i: (i, 0)),
            pl.BlockSpec((8, 8), lambda i: (0, 0)),
            pl.BlockSpec((8, C2), lambda i: (0, 0)),
        ],
        out_specs=pl.BlockSpec((R_BLK, C2), lambda i: (i, 0)),
        out_shape=jax.ShapeDtypeStruct((N_NODES, C2), jnp.float32),
    )(t2, ac0, ac1, mx2, b2b)


_sc_edge_1 = _make_sc_edge(H1, O1, 625, 4, 15, 72, 8)
_sc_edge_2 = _make_sc_edge(1, C2, 80, 7, 127, 48, 1)


def kernel(x, edge_index, W1, att_src1, att_dst1, b1, W2, att_src2, att_dst2, b2):
    src = edge_index[0].reshape(NW, N_CH, B_E)
    dst = edge_index[1].reshape(NW, N_CH, B_E)

    eye8 = jnp.eye(8, dtype=jnp.float32)
    a_src_m = (att_src1[:, :, None] * eye8[:, None, :]).reshape(64, 8)
    a_dst_m = (att_dst1[:, :, None] * eye8[:, None, :]).reshape(64, 8)
    rep = jnp.repeat(eye8, 8, axis=1)
    w2p = jnp.pad(W2, ((0, 0), (0, 1)))
    s2 = jnp.pad(att_src2.reshape(C2, 1), ((0, 1), (0, 7)))
    d2 = jnp.pad(att_dst2.reshape(C2, 1), ((0, 1), (0, 7)))
    b1b = jnp.broadcast_to(b1[None, :], (8, 64))
    b2b = jnp.broadcast_to(b2[None, :], (8, C2))

    t1, mx1 = _tc_a(x, W1, a_src_m, a_dst_m)
    mx1_bk = jnp.broadcast_to(mx1[0][:, None], (H1, RW))
    adp1 = t1[:, 8:16].reshape(625, RW)
    acc1 = _sc_edge_1(t1, src, dst, adp1, mx1_bk)
    t2, mx2 = _tc_b1(t1, acc1[0], acc1[1], mx1, b1b, rep, w2p, s2, d2)
    mx2_bk = jnp.broadcast_to(mx2[0:1, 0:1], (1, RW))
    adp2 = jnp.pad(t2[:, 8], (0, 240)).reshape(80, RW)
    acc2 = _sc_edge_2(t2, src, dst, adp2, mx2_bk)
    return _tc_b2(t2, acc2[0], acc2[1], mx2, b2b)
